# Initial kernel scaffold; baseline (speedup 1.0000x reference)
import functools

import jax
import jax.numpy as jnp
from jax import lax
from jax.experimental import pallas as pl
from jax.experimental.pallas import tpu as pltpu
from jax.experimental.pallas import tpu_sc as plsc

_B, _C, _H, _W, _K = 8, 3, 512, 512, 21
_BETA = 1.0
_HW = _H * _W
_NC, _NS, _L = 2, 16, 16
_NW = _NC * _NS
_PPW = _B * _HW // _NW
_CH = 4096
_NCHUNK = _PPW // _CH
_KP = 24
_ACC_ROWS = 2 * _KP


def _sc_body(inp_hbm, tgt_hbm, msk_hbm, out_hbm, ibuf, tbuf, mbuf, acc, sem):
    cc = lax.axis_index("c")
    ss = lax.axis_index("s")
    wid = cc * _NS + ss
    pix0 = wid * _PPW
    b = pix0 // _HW
    poff = pix0 - b * _HW

    zero = jnp.zeros((_L,), jnp.float32)
    for k in range(_ACC_ROWS):
        acc[k, :] = zero

    def _copies(g, sl):
        col0 = poff + g * _CH
        cps = []
        for c in range(_C):
            base = pl.multiple_of((b * _C + c) * _HW + col0, _CH)
            cps.append(pltpu.make_async_copy(
                inp_hbm.at[pl.ds(base, _CH)], ibuf.at[sl, c], sem))
            cps.append(pltpu.make_async_copy(
                tgt_hbm.at[pl.ds(base, _CH)], tbuf.at[sl, c], sem))
        mbase = pl.multiple_of(pix0 + g * _CH, _CH)
        cps.append(pltpu.make_async_copy(
            msk_hbm.at[pl.ds(mbase, _CH)], mbuf.at[sl], sem))
        return cps

    def _start(g, sl):
        for cp in _copies(g, sl):
            cp.start()

    def _wait(g, sl):
        for cp in _copies(g, sl):
            cp.wait()

    lanes = lax.broadcasted_iota(jnp.int32, (_L,), 0)
    ones = jnp.ones((_L,), jnp.float32)

    _start(0, 0)
    for g in range(_NCHUNK):
        sl = g % 2
        if g + 1 < _NCHUNK:
            _start(g + 1, 1 - sl)
        _wait(g, sl)

        def _it(i, carry):
            s = pl.ds(i * _L, _L)
            d = (jnp.abs(ibuf[sl, 0, s] - tbuf[sl, 0, s])
                 + jnp.abs(ibuf[sl, 1, s] - tbuf[sl, 1, s])
                 + jnp.abs(ibuf[sl, 2, s] - tbuf[sl, 2, s]))
            mv = mbuf[sl, s]
            plsc.addupdate_scatter(acc, [mv, lanes], d)
            plsc.addupdate_scatter(acc, [mv + _KP, lanes], ones)
            return carry

        lax.fori_loop(0, _CH // _L, _it, 0)

    pltpu.sync_copy(acc, out_hbm.at[wid])


_sc_segment_sums = functools.partial(
    pl.kernel,
    mesh=plsc.VectorSubcoreMesh(core_axis_name="c", subcore_axis_name="s"),
    out_type=jax.ShapeDtypeStruct((_NW, _ACC_ROWS, _L), jnp.float32),
    scratch_types=[
        pltpu.VMEM((2, _C, _CH), jnp.float32),
        pltpu.VMEM((2, _C, _CH), jnp.float32),
        pltpu.VMEM((2, _CH), jnp.int32),
        pltpu.VMEM((_ACC_ROWS, _L), jnp.float32),
        pltpu.SemaphoreType.DMA,
    ],
)(_sc_body)


def kernel(input, target, mask):
    part = _sc_segment_sums(
        input.reshape(-1), target.reshape(-1), mask.reshape(-1))
    part = part.reshape(_B, _NW // _B, _ACC_ROWS, _L).sum(axis=(1, 3))
    sums = part[:, :_K]
    cnts = part[:, _KP:_KP + _K]
    avg = sums / jnp.maximum(cnts * _C, 1.0)
    m = avg.max()
    t = (avg * sums).sum()
    total = sums.sum()
    n = _B * _C * _H * _W
    return (total + _BETA * (t / m)) / n


# speedup vs baseline: 202.5562x; 202.5562x over previous
import functools

import jax
import jax.numpy as jnp
from jax import lax
from jax.experimental import pallas as pl
from jax.experimental.pallas import tpu as pltpu
from jax.experimental.pallas import tpu_sc as plsc

_B, _C, _H, _W, _K = 8, 3, 512, 512, 21
_BETA = 1.0
_HW = _H * _W
_NC, _NS, _L = 2, 16, 16
_NW = _NC * _NS
_PPW = _B * _HW // _NW
_CH = 4096
_NCHUNK = _PPW // _CH
_KP = 24
_ACC_ROWS = 2 * _KP
_ACC_N = _ACC_ROWS * _L


def _sc_body(inp_hbm, tgt_hbm, msk_hbm, out_hbm, ibuf, tbuf, mbuf, acc, sem):
    cc = lax.axis_index("c")
    ss = lax.axis_index("s")
    wid = cc * _NS + ss
    pix0 = wid * _PPW
    b = pix0 // _HW
    poff = pix0 - b * _HW

    zero = jnp.zeros((_L,), jnp.float32)
    for k in range(_ACC_ROWS):
        acc[pl.ds(k * _L, _L)] = zero

    def _copies(g, sl):
        col0 = poff + g * _CH
        cps = []
        for c in range(_C):
            base = pl.multiple_of((b * _C + c) * _HW + col0, _CH)
            dst = pl.ds((sl * _C + c) * _CH, _CH)
            cps.append(pltpu.make_async_copy(
                inp_hbm.at[pl.ds(base, _CH)], ibuf.at[dst], sem))
            cps.append(pltpu.make_async_copy(
                tgt_hbm.at[pl.ds(base, _CH)], tbuf.at[dst], sem))
        mbase = pl.multiple_of(pix0 + g * _CH, _CH)
        cps.append(pltpu.make_async_copy(
            msk_hbm.at[pl.ds(mbase, _CH)], mbuf.at[pl.ds(sl * _CH, _CH)], sem))
        return cps

    def _start(g, sl):
        for cp in _copies(g, sl):
            cp.start()

    def _wait(g, sl):
        for cp in _copies(g, sl):
            cp.wait()

    lanes = lax.broadcasted_iota(jnp.int32, (_L,), 0)
    ones = jnp.ones((_L,), jnp.float32)

    _start(0, 0)
    for g in range(_NCHUNK):
        sl = g % 2
        if g + 1 < _NCHUNK:
            _start(g + 1, 1 - sl)
        _wait(g, sl)

        def _it(i, carry):
            off = i * _L
            d = jnp.zeros((_L,), jnp.float32)
            for c in range(_C):
                cbase = (sl * _C + c) * _CH
                d = d + jnp.abs(ibuf[pl.ds(cbase + off, _L)]
                                - tbuf[pl.ds(cbase + off, _L)])
            mv = mbuf[pl.ds(sl * _CH + off, _L)]
            idx = mv * _L + lanes
            plsc.addupdate_scatter(acc, [idx], d)
            plsc.addupdate_scatter(acc, [idx + _KP * _L], ones)
            return carry

        lax.fori_loop(0, _CH // _L, _it, 0)

    pltpu.sync_copy(acc, out_hbm.at[pl.ds(wid * _ACC_N, _ACC_N)])


_sc_segment_sums = functools.partial(
    pl.kernel,
    mesh=plsc.VectorSubcoreMesh(core_axis_name="c", subcore_axis_name="s"),
    out_type=jax.ShapeDtypeStruct((_NW * _ACC_N,), jnp.float32),
    compiler_params=pltpu.CompilerParams(needs_layout_passes=False),
    scratch_types=[
        pltpu.VMEM((2 * _C * _CH,), jnp.float32),
        pltpu.VMEM((2 * _C * _CH,), jnp.float32),
        pltpu.VMEM((2 * _CH,), jnp.int32),
        pltpu.VMEM((_ACC_N,), jnp.float32),
        pltpu.SemaphoreType.DMA,
    ],
)(_sc_body)


def kernel(input, target, mask):
    part = _sc_segment_sums(
        input.reshape(-1), target.reshape(-1), mask.reshape(-1))
    part = part.reshape(_B, _NW // _B, _ACC_ROWS, _L).sum(axis=(1, 3))
    sums = part[:, :_K]
    cnts = part[:, _KP:_KP + _K]
    avg = sums / jnp.maximum(cnts * _C, 1.0)
    m = avg.max()
    t = (avg * sums).sum()
    total = sums.sum()
    n = _B * _C * _H * _W
    return (total + _BETA * (t / m)) / n


# speedup vs baseline: 221.8545x; 1.0953x over previous
import functools

import jax
import jax.numpy as jnp
from jax import lax
from jax.experimental import pallas as pl
from jax.experimental.pallas import tpu as pltpu
from jax.experimental.pallas import tpu_sc as plsc

_B, _C, _H, _W, _K = 8, 3, 512, 512, 21
_BETA = 1.0
_HW = _H * _W
_NC, _NS, _L = 2, 16, 16
_NW = _NC * _NS
_PPW = _B * _HW // _NW
_CH = 4096
_NCHUNK = _PPW // _CH
_KP = 24
_ACC_ROWS = 2 * _KP
_ACC_N = _ACC_ROWS * _L


def _sc_body(inp_hbm, tgt_hbm, msk_hbm, out_hbm, ibuf, tbuf, mbuf, acc, sem):
    cc = lax.axis_index("c")
    ss = lax.axis_index("s")
    wid = cc * _NS + ss
    pix0 = wid * _PPW
    b = pix0 // _HW
    poff = pix0 - b * _HW

    zero = jnp.zeros((_L,), jnp.float32)
    for k in range(_ACC_ROWS):
        acc[pl.ds(k * _L, _L)] = zero

    def _copies(g, sl):
        col0 = poff + g * _CH
        cps = []
        for c in range(_C):
            base = pl.multiple_of((b * _C + c) * _HW + col0, _CH)
            dst = pl.ds((sl * _C + c) * _CH, _CH)
            cps.append(pltpu.make_async_copy(
                inp_hbm.at[pl.ds(base, _CH)], ibuf.at[dst], sem))
            cps.append(pltpu.make_async_copy(
                tgt_hbm.at[pl.ds(base, _CH)], tbuf.at[dst], sem))
        mbase = pl.multiple_of(pix0 + g * _CH, _CH)
        cps.append(pltpu.make_async_copy(
            msk_hbm.at[pl.ds(mbase, _CH)], mbuf.at[pl.ds(sl * _CH, _CH)], sem))
        return cps

    def _start(g, sl):
        for cp in _copies(g, sl):
            cp.start()

    def _wait(g, sl):
        for cp in _copies(g, sl):
            cp.wait()

    lanes = lax.broadcasted_iota(jnp.int32, (_L,), 0)
    ones = jnp.ones((_L,), jnp.float32)

    _start(0, 0)
    for g in range(_NCHUNK):
        sl = g % 2
        if g + 1 < _NCHUNK:
            _start(g + 1, 1 - sl)
        _wait(g, sl)

        @plsc.parallel_loop(0, _CH // _L, unroll=8)
        def _it(i):
            off = i * _L
            d = jnp.zeros((_L,), jnp.float32)
            for c in range(_C):
                cbase = (sl * _C + c) * _CH
                d = d + jnp.abs(ibuf[pl.ds(cbase + off, _L)]
                                - tbuf[pl.ds(cbase + off, _L)])
            mv = mbuf[pl.ds(sl * _CH + off, _L)]
            idx = mv * _L + lanes
            plsc.addupdate_scatter(acc, [idx], d)
            plsc.addupdate_scatter(acc, [idx + _KP * _L], ones)

    pltpu.sync_copy(acc, out_hbm.at[pl.ds(wid * _ACC_N, _ACC_N)])


_sc_segment_sums = functools.partial(
    pl.kernel,
    mesh=plsc.VectorSubcoreMesh(core_axis_name="c", subcore_axis_name="s"),
    out_type=jax.ShapeDtypeStruct((_NW * _ACC_N,), jnp.float32),
    compiler_params=pltpu.CompilerParams(needs_layout_passes=False),
    scratch_types=[
        pltpu.VMEM((2 * _C * _CH,), jnp.float32),
        pltpu.VMEM((2 * _C * _CH,), jnp.float32),
        pltpu.VMEM((2 * _CH,), jnp.int32),
        pltpu.VMEM((_ACC_N,), jnp.float32),
        pltpu.SemaphoreType.DMA,
    ],
)(_sc_body)


def kernel(input, target, mask):
    part = _sc_segment_sums(
        input.reshape(-1), target.reshape(-1), mask.reshape(-1))
    part = part.reshape(_B, _NW // _B, _ACC_ROWS, _L).sum(axis=(1, 3))
    sums = part[:, :_K]
    cnts = part[:, _KP:_KP + _K]
    avg = sums / jnp.maximum(cnts * _C, 1.0)
    m = avg.max()
    t = (avg * sums).sum()
    total = sums.sum()
    n = _B * _C * _H * _W
    return (total + _BETA * (t / m)) / n


# speedup vs baseline: 439.7129x; 1.9820x over previous
import functools

import jax
import jax.numpy as jnp
from jax import lax
from jax.experimental import pallas as pl
from jax.experimental.pallas import tpu as pltpu
from jax.experimental.pallas import tpu_sc as plsc

_B, _C, _H, _W, _K = 8, 3, 512, 512, 21
_BETA = 1.0
_HW = _H * _W
_NC, _NS, _L = 2, 16, 16
_NW = _NC * _NS
_KP = 24
_ACC_ROWS = 2 * _KP
_ACC_N = _ACC_ROWS * _L
_RCH = 8
_CH = _RCH * _W
_RSUB = _H // 4
_NCHUNK = _RSUB // _RCH
_JGRP = _W // _L


def _sc_body(inp_hbm, tgt_hbm, msk_hbm, out_hbm,
             i0a, i1a, i2a, i0b, i1b, i2b,
             t0a, t1a, t2a, t0b, t1b, t2b,
             ma, mb, acc, sem):
    ibufs = ((i0a, i1a, i2a), (i0b, i1b, i2b))
    tbufs = ((t0a, t1a, t2a), (t0b, t1b, t2b))
    mbufs = (ma, mb)

    cc = lax.axis_index("c")
    ss = lax.axis_index("s")
    wid = cc * _NS + ss
    b = wid // 4
    r0 = (wid % 4) * _RSUB

    zero = jnp.zeros((_L,), jnp.float32)
    for k in range(_ACC_ROWS):
        acc[pl.ds(k * _L, _L)] = zero

    def _copies(g, sl):
        h0 = pl.multiple_of(r0 + g * _RCH, _RCH)
        cps = []
        for c in range(_C):
            cps.append(pltpu.make_async_copy(
                inp_hbm.at[b, c, pl.ds(h0, _RCH), :], ibufs[sl][c], sem))
            cps.append(pltpu.make_async_copy(
                tgt_hbm.at[b, c, pl.ds(h0, _RCH), :], tbufs[sl][c], sem))
        cps.append(pltpu.make_async_copy(
            msk_hbm.at[b, pl.ds(h0, _RCH), :], mbufs[sl], sem))
        return cps

    def _start(g, sl):
        for cp in _copies(g, sl):
            cp.start()

    def _wait(g, sl):
        for cp in _copies(g, sl):
            cp.wait()

    lanes = lax.broadcasted_iota(jnp.int32, (_L,), 0)
    ones = jnp.ones((_L,), jnp.float32)

    _start(0, 0)
    for g in range(_NCHUNK):
        sl = g % 2
        if g + 1 < _NCHUNK:
            _start(g + 1, 1 - sl)
        _wait(g, sl)

        ib, tb, mb_ = ibufs[sl], tbufs[sl], mbufs[sl]

        @plsc.parallel_loop(0, _RCH * _JGRP, unroll=8)
        def _it(t):
            i = lax.shift_right_logical(t, 5)
            j = lax.shift_left(lax.bitwise_and(t, _JGRP - 1), 4)
            d = (jnp.abs(ib[0][i, pl.ds(j, _L)] - tb[0][i, pl.ds(j, _L)])
                 + jnp.abs(ib[1][i, pl.ds(j, _L)] - tb[1][i, pl.ds(j, _L)])
                 + jnp.abs(ib[2][i, pl.ds(j, _L)] - tb[2][i, pl.ds(j, _L)]))
            mv = mb_[i, pl.ds(j, _L)]
            idx = mv * _L + lanes
            plsc.addupdate_scatter(acc, [idx], d)
            plsc.addupdate_scatter(acc, [idx + _KP * _L], ones)

    pltpu.sync_copy(acc, out_hbm.at[pl.ds(wid * _ACC_N, _ACC_N)])


_sc_segment_sums = functools.partial(
    pl.kernel,
    mesh=plsc.VectorSubcoreMesh(core_axis_name="c", subcore_axis_name="s"),
    out_type=jax.ShapeDtypeStruct((_NW * _ACC_N,), jnp.float32),
    compiler_params=pltpu.CompilerParams(needs_layout_passes=False),
    scratch_types=(
        [pltpu.VMEM((_RCH, _W), jnp.float32) for _ in range(12)]
        + [pltpu.VMEM((_RCH, _W), jnp.int32) for _ in range(2)]
        + [pltpu.VMEM((_ACC_N,), jnp.float32), pltpu.SemaphoreType.DMA]
    ),
)(_sc_body)


def kernel(input, target, mask):
    part = _sc_segment_sums(input, target, mask)
    part = part.reshape(_B, _NW // _B, _ACC_ROWS, _L).sum(axis=(1, 3))
    sums = part[:, :_K]
    cnts = part[:, _KP:_KP + _K]
    avg = sums / jnp.maximum(cnts * _C, 1.0)
    m = avg.max()
    t = (avg * sums).sum()
    total = sums.sum()
    n = _B * _C * _H * _W
    return (total + _BETA * (t / m)) / n


# speedup vs baseline: 448.1934x; 1.0193x over previous
import functools

import jax
import jax.numpy as jnp
from jax import lax
from jax.experimental import pallas as pl
from jax.experimental.pallas import tpu as pltpu
from jax.experimental.pallas import tpu_sc as plsc

_B, _C, _H, _W, _K = 8, 3, 512, 512, 21
_BETA = 1.0
_HW = _H * _W
_NC, _NS, _L = 2, 16, 16
_NW = _NC * _NS
_KP = 24
_ACC_ROWS = 2 * _KP
_ACC_N = _ACC_ROWS * _L
_RCH = 8
_CH = _RCH * _W
_RSUB = _H // 4
_NCHUNK = _RSUB // _RCH
_JGRP = _W // _L


def _sc_body(inp_hbm, tgt_hbm, msk_hbm, out_hbm,
             ia, ib_, ta, tb_, ma, mb_, acc, sem):
    ibufs = (ia, ib_)
    tbufs = (ta, tb_)
    mbufs = (ma, mb_)

    cc = lax.axis_index("c")
    ss = lax.axis_index("s")
    wid = cc * _NS + ss
    b = wid // 4
    r0 = (wid % 4) * _RSUB

    zero = jnp.zeros((_L,), jnp.float32)
    for k in range(_ACC_ROWS):
        acc[pl.ds(k * _L, _L)] = zero

    def _copies(g, sl):
        h0 = pl.multiple_of(r0 + g * _RCH, _RCH)
        return [
            pltpu.make_async_copy(
                inp_hbm.at[b, :, pl.ds(h0, _RCH), :], ibufs[sl], sem),
            pltpu.make_async_copy(
                tgt_hbm.at[b, :, pl.ds(h0, _RCH), :], tbufs[sl], sem),
            pltpu.make_async_copy(
                msk_hbm.at[b, pl.ds(h0, _RCH), :], mbufs[sl], sem),
        ]

    def _start(g, sl):
        for cp in _copies(g, sl):
            cp.start()

    def _wait(g, sl):
        for cp in _copies(g, sl):
            cp.wait()

    lanes = lax.broadcasted_iota(jnp.int32, (_L,), 0)
    ones = jnp.ones((_L,), jnp.float32)

    _start(0, 0)
    for g in range(_NCHUNK):
        sl = g % 2
        if g + 1 < _NCHUNK:
            _start(g + 1, 1 - sl)
        _wait(g, sl)

        ibc, tbc, mbc = ibufs[sl], tbufs[sl], mbufs[sl]

        @plsc.parallel_loop(0, _RCH * _JGRP, unroll=8)
        def _it(t):
            i = lax.shift_right_logical(t, 5)
            j = lax.shift_left(lax.bitwise_and(t, _JGRP - 1), 4)
            d = (jnp.abs(ibc[0, i, pl.ds(j, _L)] - tbc[0, i, pl.ds(j, _L)])
                 + jnp.abs(ibc[1, i, pl.ds(j, _L)] - tbc[1, i, pl.ds(j, _L)])
                 + jnp.abs(ibc[2, i, pl.ds(j, _L)] - tbc[2, i, pl.ds(j, _L)]))
            mv = mbc[i, pl.ds(j, _L)]
            idx = mv * _L + lanes
            plsc.addupdate_scatter(acc, [idx], d)
            plsc.addupdate_scatter(acc, [idx + _KP * _L], ones)

    pltpu.sync_copy(acc, out_hbm.at[pl.ds(wid * _ACC_N, _ACC_N)])


_sc_segment_sums = functools.partial(
    pl.kernel,
    mesh=plsc.VectorSubcoreMesh(core_axis_name="c", subcore_axis_name="s"),
    out_type=jax.ShapeDtypeStruct((_NW * _ACC_N,), jnp.float32),
    compiler_params=pltpu.CompilerParams(needs_layout_passes=False),
    scratch_types=(
        [pltpu.VMEM((_C, _RCH, _W), jnp.float32) for _ in range(4)]
        + [pltpu.VMEM((_RCH, _W), jnp.int32) for _ in range(2)]
        + [pltpu.VMEM((_ACC_N,), jnp.float32), pltpu.SemaphoreType.DMA]
    ),
)(_sc_body)


def kernel(input, target, mask):
    part = _sc_segment_sums(input, target, mask)
    part = part.reshape(_B, _NW // _B, _ACC_ROWS, _L).sum(axis=(1, 3))
    sums = part[:, :_K]
    cnts = part[:, _KP:_KP + _K]
    avg = sums / jnp.maximum(cnts * _C, 1.0)
    m = avg.max()
    t = (avg * sums).sum()
    total = sums.sum()
    n = _B * _C * _H * _W
    return (total + _BETA * (t / m)) / n


# speedup vs baseline: 483.6434x; 1.0791x over previous
import functools

import jax
import jax.numpy as jnp
from jax import lax
from jax.experimental import pallas as pl
from jax.experimental.pallas import tpu as pltpu
from jax.experimental.pallas import tpu_sc as plsc

_B, _C, _H, _W, _K = 8, 3, 512, 512, 21
_BETA = 1.0
_HW = _H * _W
_NC, _NS, _L = 2, 16, 16
_NW = _NC * _NS
_KP = 24
_ACC_ROWS = 2 * _KP
_ACC_N = _ACC_ROWS * _L
_RCH = 16
_CH = _RCH * _W
_RSUB = _H // 4
_NCHUNK = _RSUB // _RCH
_JGRP = _W // _L


def _sc_body(inp_hbm, tgt_hbm, msk_hbm, out_hbm,
             ia, ib_, ta, tb_, ma, mb_, acc, sem):
    ibufs = (ia, ib_)
    tbufs = (ta, tb_)
    mbufs = (ma, mb_)

    cc = lax.axis_index("c")
    ss = lax.axis_index("s")
    wid = cc * _NS + ss
    b = wid // 4
    r0 = (wid % 4) * _RSUB

    zero = jnp.zeros((_L,), jnp.float32)
    for k in range(_ACC_ROWS):
        acc[pl.ds(k * _L, _L)] = zero

    def _copies(g, sl):
        h0 = pl.multiple_of(r0 + g * _RCH, _RCH)
        return [
            pltpu.make_async_copy(
                inp_hbm.at[b, :, pl.ds(h0, _RCH), :], ibufs[sl], sem),
            pltpu.make_async_copy(
                tgt_hbm.at[b, :, pl.ds(h0, _RCH), :], tbufs[sl], sem),
            pltpu.make_async_copy(
                msk_hbm.at[b, pl.ds(h0, _RCH), :], mbufs[sl], sem),
        ]

    def _start(g, sl):
        for cp in _copies(g, sl):
            cp.start()

    def _wait(g, sl):
        for cp in _copies(g, sl):
            cp.wait()

    lanes = lax.broadcasted_iota(jnp.int32, (_L,), 0)
    ones = jnp.ones((_L,), jnp.float32)

    _start(0, 0)
    for g in range(_NCHUNK):
        sl = g % 2
        if g + 1 < _NCHUNK:
            _start(g + 1, 1 - sl)
        _wait(g, sl)

        ibc, tbc, mbc = ibufs[sl], tbufs[sl], mbufs[sl]

        @plsc.parallel_loop(0, _RCH * _JGRP, unroll=8)
        def _it(t):
            i = lax.shift_right_logical(t, 5)
            j = lax.shift_left(lax.bitwise_and(t, _JGRP - 1), 4)
            d = (jnp.abs(ibc[0, i, pl.ds(j, _L)] - tbc[0, i, pl.ds(j, _L)])
                 + jnp.abs(ibc[1, i, pl.ds(j, _L)] - tbc[1, i, pl.ds(j, _L)])
                 + jnp.abs(ibc[2, i, pl.ds(j, _L)] - tbc[2, i, pl.ds(j, _L)]))
            mv = mbc[i, pl.ds(j, _L)]
            idx = mv * _L + lanes
            plsc.addupdate_scatter(acc, [idx], d)
            plsc.addupdate_scatter(acc, [idx + _KP * _L], ones)

    pltpu.sync_copy(acc, out_hbm.at[pl.ds(wid * _ACC_N, _ACC_N)])


_sc_segment_sums = functools.partial(
    pl.kernel,
    mesh=plsc.VectorSubcoreMesh(core_axis_name="c", subcore_axis_name="s"),
    out_type=jax.ShapeDtypeStruct((_NW * _ACC_N,), jnp.float32),
    compiler_params=pltpu.CompilerParams(needs_layout_passes=False),
    scratch_types=(
        [pltpu.VMEM((_C, _RCH, _W), jnp.float32) for _ in range(4)]
        + [pltpu.VMEM((_RCH, _W), jnp.int32) for _ in range(2)]
        + [pltpu.VMEM((_ACC_N,), jnp.float32), pltpu.SemaphoreType.DMA]
    ),
)(_sc_body)


def kernel(input, target, mask):
    part = _sc_segment_sums(input, target, mask)
    part = part.reshape(_B, _NW // _B, _ACC_ROWS, _L).sum(axis=(1, 3))
    sums = part[:, :_K]
    cnts = part[:, _KP:_KP + _K]
    avg = sums / jnp.maximum(cnts * _C, 1.0)
    m = avg.max()
    t = (avg * sums).sum()
    total = sums.sum()
    n = _B * _C * _H * _W
    return (total + _BETA * (t / m)) / n
